# baseline (speedup 1.0000x reference)
import functools

import jax
import jax.numpy as jnp
from jax import lax
from jax.experimental import pallas as pl
from jax.experimental.pallas import tpu as pltpu
from jax.experimental.pallas import tpu_sc as plsc

NC = 2
NS = 16
LANES = 16
NW = NC * NS

DIM = 64
PAIR = 2 * DIM
NEGS = 20
SB = 128
GROUPS = SB // LANES
DUNROLL = 8


def _iota16():
    return lax.iota(jnp.int32, LANES)


def _log_sigmoid(x):
    u = jnp.exp(-jnp.abs(x))
    z = u / (u + 2.0)
    z2 = z * z
    p = 1.0 + z2 * (1.0 / 3.0 + z2 * (0.2 + z2 * (1.0 / 7.0)))
    return jnp.minimum(x, 0.0) - 2.0 * z * p


def _gather16(ref, rows, cols):
    return plsc.load_gather(ref, [rows, cols])


def _dot_group(a_ref, a_rows, a_coff, b_ref, b_rows, b_coff):
    zero = jnp.zeros((LANES,), jnp.float32)

    def body(i, accs):
        d0 = i * DUNROLL
        out = []
        for j in range(DUNROLL):
            av = _gather16(a_ref, a_rows, a_coff + (d0 + j))
            bv = _gather16(b_ref, b_rows, b_coff + (d0 + j))
            out.append(accs[j] + av * bv)
        return tuple(out)

    accs = lax.fori_loop(0, DIM // DUNROLL, body, (zero,) * DUNROLL)
    r = accs[0]
    for a in accs[1:]:
        r = r + a
    return r


def _gather_rows(table_r, idx_view, dst, sem):
    return pltpu.async_copy(table_r.at[idx_view], dst, sem)


def _gather16_1d(ref, rows):
    return plsc.load_gather(ref, [rows])


def _pair_split(idx_ref, off, rows):
    v = _gather16_1d(idx_ref, off + rows)
    return (v & 1) * DIM


def _make_sc_call(batch):
    rpw = batch // NW
    nsb = rpw // SB

    def body(center_r, context_r, negflat_r, in_pair_r, out_pair_r,
             out_r, idx_ctr, idx_pos, idx_neg, gidx_ctr, gidx_pos, gidx_neg,
             ctr_buf, pos_buf, neg_buf, stage, sem):
        wid = lax.axis_index("s") * NC + lax.axis_index("c")
        base = pl.multiple_of(wid * rpw, 8)
        pltpu.sync_copy(center_r.at[pl.ds(base, rpw)], idx_ctr)
        pltpu.sync_copy(context_r.at[pl.ds(base, rpw)], idx_pos)

        def shift_chunk(i, src_ref, dst_ref):
            sl = pl.ds(i * LANES, LANES)
            dst_ref[sl] = lax.shift_right_logical(src_ref[sl], 1)

        def shift_all(i, _):
            shift_chunk(i, idx_ctr, gidx_ctr)
            shift_chunk(i, idx_pos, gidx_pos)
            return 0
        lax.fori_loop(0, rpw // LANES, shift_all, 0)

        def sbody(sb, tot):
            off = pl.multiple_of(sb * SB, 8)
            pltpu.sync_copy(
                negflat_r.at[pl.ds((base + off) * NEGS, SB * NEGS)], idx_neg)
            lax.fori_loop(
                0, (SB * NEGS) // LANES,
                lambda i, _: (shift_chunk(i, idx_neg, gidx_neg), 0)[1], 0)
            c1 = _gather_rows(in_pair_r, gidx_ctr.at[pl.ds(off, SB)], ctr_buf,
                              sem)
            c2 = _gather_rows(out_pair_r, gidx_pos.at[pl.ds(off, SB)], pos_buf,
                              sem)
            c1.wait()
            c2.wait()
            for g in range(GROUPS):
                rows = g * LANES + _iota16()
                pc = _pair_split(idx_ctr, off, rows)
                pp = _pair_split(idx_pos, off, rows)
                s = _dot_group(ctr_buf, rows, pc, pos_buf, rows, pp)
                tot = tot + _log_sigmoid(s)

            def kbody(k, t):
                koff = pl.multiple_of(k * SB, 8)
                _gather_rows(out_pair_r, gidx_neg.at[pl.ds(koff, SB)], neg_buf,
                             sem).wait()
                for g in range(GROUPS):
                    rowsn = g * LANES + _iota16()
                    flat = koff + rowsn
                    rowsc = flat // NEGS
                    pc = _pair_split(idx_ctr, off, rowsc)
                    pn = _pair_split(idx_neg, 0, flat)
                    s = _dot_group(ctr_buf, rowsc, pc, neg_buf, rowsn, pn)
                    t = t + _log_sigmoid(-s)
                return t

            return lax.fori_loop(0, NEGS, kbody, tot)

        tot = lax.fori_loop(0, nsb, sbody, jnp.zeros((LANES,), jnp.float32))
        stage[...] = tot * (-1.0 / batch)
        pltpu.sync_copy(stage, out_r.at[wid])

    mesh = plsc.VectorSubcoreMesh(
        core_axis_name="c", subcore_axis_name="s",
        num_cores=NC, num_subcores=NS)
    return pl.kernel(
        body,
        out_type=jax.ShapeDtypeStruct((NW, LANES), jnp.float32),
        mesh=mesh,
        compiler_params=pltpu.CompilerParams(
            needs_layout_passes=False, use_tc_tiling_on_sc=True),
        scratch_types=[
            pltpu.VMEM((rpw,), jnp.int32),
            pltpu.VMEM((rpw,), jnp.int32),
            pltpu.VMEM((SB * NEGS,), jnp.int32),
            pltpu.VMEM((rpw,), jnp.int32),
            pltpu.VMEM((rpw,), jnp.int32),
            pltpu.VMEM((SB * NEGS,), jnp.int32),
            pltpu.VMEM((SB, PAIR), jnp.float32),
            pltpu.VMEM((SB, PAIR), jnp.float32),
            pltpu.VMEM((SB, PAIR), jnp.float32),
            pltpu.VMEM((LANES,), jnp.float32),
            pltpu.SemaphoreType.DMA,
        ],
    )


@jax.jit
def kernel(center_words, context_words, neg_words, in_embed, out_embed):
    batch = center_words.shape[0]
    call = _make_sc_call(batch)
    partials = call(
        center_words.astype(jnp.int32),
        context_words.astype(jnp.int32),
        neg_words.reshape(-1).astype(jnp.int32),
        in_embed.reshape(-1, PAIR),
        out_embed.reshape(-1, PAIR),
    )
    return jnp.sum(partials)


# speedup vs baseline: 1.0816x; 1.0816x over previous
import functools

import jax
import jax.numpy as jnp
from jax import lax
from jax.experimental import pallas as pl
from jax.experimental.pallas import tpu as pltpu
from jax.experimental.pallas import tpu_sc as plsc

NC = 2
NS = 16
LANES = 16
NW = NC * NS

DIM = 64
NEGS = 20
CHUNK = 128
CGROUPS = CHUNK // LANES
DUNROLL = 8


def _iota16():
    return lax.iota(jnp.int32, LANES)


def _log_sigmoid(x):
    u = jnp.exp(-jnp.abs(x))
    z = u / (u + 2.0)
    z2 = z * z
    p = 1.0 + z2 * (1.0 / 3.0 + z2 * (0.2 + z2 * (1.0 / 7.0)))
    return jnp.minimum(x, 0.0) - 2.0 * z * p


def _gather16(ref, rows, cols):
    return plsc.load_gather(ref, [rows, cols])


def _dot_group(a_ref, a_rows, b_ref, b_rows):
    zero = jnp.zeros((LANES,), jnp.float32)

    def body(i, accs):
        d0 = i * DUNROLL
        out = []
        for j in range(DUNROLL):
            col = jnp.full((LANES,), d0 + j, jnp.int32)
            av = _gather16(a_ref, a_rows, col)
            bv = _gather16(b_ref, b_rows, col)
            out.append(accs[j] + av * bv)
        return tuple(out)

    accs = lax.fori_loop(0, DIM // DUNROLL, body, (zero,) * DUNROLL)
    r = accs[0]
    for a in accs[1:]:
        r = r + a
    return r


def _gather_rows(table_r, idx_view, dst, sem):
    return pltpu.async_copy(table_r.at[idx_view], dst, sem)


def _make_sc_call(batch):
    rpw = batch // NW
    nneg = (rpw * NEGS) // CHUNK

    def body(center_r, context_r, negflat_r, in_emb_r, out_emb_r,
             out_r, idx_ctr, idx_pos, idx_neg, ctr_all, pos_all,
             neg0, neg1, stage, semp, sem0, sem1):
        wid = lax.axis_index("s") * NC + lax.axis_index("c")
        base = pl.multiple_of(wid * rpw, 8)
        pltpu.sync_copy(center_r.at[pl.ds(base, rpw)], idx_ctr)
        pltpu.sync_copy(context_r.at[pl.ds(base, rpw)], idx_pos)
        pltpu.sync_copy(negflat_r.at[pl.ds(base * NEGS, rpw * NEGS)], idx_neg)

        cps = []
        for c in range(rpw // CHUNK):
            sl = pl.ds(c * CHUNK, CHUNK)
            cps.append(_gather_rows(in_emb_r, idx_ctr.at[sl],
                                    ctr_all.at[sl, :], semp))
            cps.append(_gather_rows(out_emb_r, idx_pos.at[sl],
                                    pos_all.at[sl, :], semp))

        def fire_neg(k, dst, sem):
            koff = pl.multiple_of(k * CHUNK, 8)
            return _gather_rows(out_emb_r, idx_neg.at[pl.ds(koff, CHUNK)],
                                dst, sem)

        fire_neg(0, neg0, sem0)
        fire_neg(1, neg1, sem1)
        for cp in cps:
            cp.wait()

        def pbody(g, tot):
            rows = g * LANES + _iota16()
            s = _dot_group(ctr_all, rows, pos_all, rows)
            return tot + _log_sigmoid(s)
        tot = lax.fori_loop(0, rpw // LANES, pbody,
                            jnp.zeros((LANES,), jnp.float32))

        def kbody(i, t):
            for slot, (nbuf, sem) in enumerate(((neg0, sem0), (neg1, sem1))):
                k = 2 * i + slot
                koff = pl.multiple_of(k * CHUNK, 8)
                _gather_rows(out_emb_r, idx_neg.at[pl.ds(koff, CHUNK)], nbuf,
                             sem).wait()
                for g in range(CGROUPS):
                    flat = koff + g * LANES + _iota16()
                    rowsn = g * LANES + _iota16()
                    rowsc = flat // NEGS
                    s = _dot_group(ctr_all, rowsc, nbuf, rowsn)
                    t = t + _log_sigmoid(-s)
                @pl.when(k + 2 < nneg)
                def _():
                    fire_neg(k + 2, nbuf, sem)
            return t

        tot = lax.fori_loop(0, nneg // 2, kbody, tot)
        stage[...] = tot * (-1.0 / batch)
        pltpu.sync_copy(stage, out_r.at[wid])

    mesh = plsc.VectorSubcoreMesh(
        core_axis_name="c", subcore_axis_name="s",
        num_cores=NC, num_subcores=NS)
    return pl.kernel(
        body,
        out_type=jax.ShapeDtypeStruct((NW, LANES), jnp.float32),
        mesh=mesh,
        compiler_params=pltpu.CompilerParams(
            needs_layout_passes=False, use_tc_tiling_on_sc=False),
        scratch_types=[
            pltpu.VMEM((rpw,), jnp.int32),
            pltpu.VMEM((rpw,), jnp.int32),
            pltpu.VMEM((rpw * NEGS,), jnp.int32),
            pltpu.VMEM((rpw, DIM), jnp.float32),
            pltpu.VMEM((rpw, DIM), jnp.float32),
            pltpu.VMEM((CHUNK, DIM), jnp.float32),
            pltpu.VMEM((CHUNK, DIM), jnp.float32),
            pltpu.VMEM((LANES,), jnp.float32),
            pltpu.SemaphoreType.DMA,
            pltpu.SemaphoreType.DMA,
            pltpu.SemaphoreType.DMA,
        ],
    )


@jax.jit
def kernel(center_words, context_words, neg_words, in_embed, out_embed):
    batch = center_words.shape[0]
    call = _make_sc_call(batch)
    partials = call(
        center_words.astype(jnp.int32),
        context_words.astype(jnp.int32),
        neg_words.reshape(-1).astype(jnp.int32),
        in_embed,
        out_embed,
    )
    return jnp.sum(partials)
